# baseline (speedup 1.0000x reference)
import jax
import jax.numpy as jnp
from jax.experimental import pallas as pl

H = 8
D = 128
B = 512
NS = 10000
NF = 10000
NC = 10000
E = 160000


def _gat(xs, xd, src, dst, nd, p):
    hs = (xs @ p["W"]).reshape(-1, H, D)
    hd = (xd @ p["W"]).reshape(-1, H, D)
    el = (hs * p["al"][None]).sum(-1)
    er = (hd * p["ar"][None]).sum(-1)
    e = el[src] + er[dst]
    e = jnp.where(e >= 0, e, 0.2 * e)
    m = jax.ops.segment_max(e, dst, num_segments=nd)
    m = jnp.where(jnp.isfinite(m), m, 0.0)
    ex = jnp.exp(e - m[dst])
    den = jax.ops.segment_sum(ex, dst, num_segments=nd)
    alpha = ex / den[dst]
    rst = jax.ops.segment_sum(alpha[:, :, None] * hs[src], dst, num_segments=nd)
    rst = rst + (xd @ p["rW"]).reshape(-1, H, D) + p["b"].reshape(1, H, D)
    return rst


def _seg_mean_broadcast(x, seg):
    s = jax.ops.segment_sum(x, seg, num_segments=B)
    c = jax.ops.segment_sum(jnp.ones((x.shape[0],), x.dtype), seg, num_segments=B)
    return (s / jnp.maximum(c, 1.0)[:, None])[seg]


def _combine_body(a_ref, mean_ref, o_ref):
    o_ref[...] = jnp.max(a_ref[...], axis=1) + mean_ref[...]


def _combine(a, mean):
    n = a.shape[0]
    blk = 1000
    return pl.pallas_call(
        _combine_body,
        grid=(n // blk,),
        in_specs=[
            pl.BlockSpec((blk, H, D), lambda i: (i, 0, 0)),
            pl.BlockSpec((blk, D), lambda i: (i, 0)),
        ],
        out_specs=pl.BlockSpec((blk, D), lambda i: (i, 0)),
        out_shape=jax.ShapeDtypeStruct((n, D), jnp.float32),
    )(a, mean)


def kernel(feat_s1, feat_f1, feat_c1, edge_intra, edge_inter, seg_s1, seg_f1, seg_c1, params):
    h1_s1 = _gat(feat_s1, feat_s1, edge_intra[0], edge_intra[1], NS, params["c1_intra"])
    h1_f1 = _gat(feat_s1, feat_f1, edge_inter[0], edge_inter[1], NF, params["c1_inter"])
    h2_s1 = _gat(feat_s1, feat_s1, edge_intra[1], edge_intra[0], NS, params["c2_intra"]) \
        + _gat(feat_f1, feat_s1, edge_inter[1], edge_inter[0], NS, params["c2_inter"])
    h_f1 = _combine(h1_f1, _seg_mean_broadcast(feat_f1, seg_f1))
    h_c1 = _seg_mean_broadcast(feat_c1, seg_c1)
    h_s1 = _combine(h1_s1 + h2_s1, _seg_mean_broadcast(feat_s1, seg_s1))
    return h_f1, h_c1, h_s1


# speedup vs baseline: 10.5031x; 10.5031x over previous
import functools

import jax
import jax.numpy as jnp
from jax import lax
from jax.experimental import pallas as pl
from jax.experimental.pallas import tpu as pltpu
from jax.experimental.pallas import tpu_sc as plsc

H = 8
D = 128
B = 512
NS = 10000
E = 160000
NG = 4
K = 128
NBLKE = E // K
NP = 10240
NPT = NP // 16
NBLKN = NS // K
NTAIL = NS - NBLKN * K
HS_ROWS = NS * H


def _sc_edge_kernel():
    mesh = plsc.VectorSubcoreMesh(core_axis_name="c", subcore_axis_name="s")
    f32 = jnp.float32
    i32 = jnp.int32

    out_type = [
        jax.ShapeDtypeStruct((H * NP, D), f32),
        jax.ShapeDtypeStruct((H * NP, D), f32),
        jax.ShapeDtypeStruct((3 * NS, D), f32),
        jax.ShapeDtypeStruct((2 * 16 * NP,), f32),
        jax.ShapeDtypeStruct((2 * NP,), f32),
        jax.ShapeDtypeStruct((2 * E,), f32),
        jax.ShapeDtypeStruct((3 * B, D), f32),
    ]
    scratch_types = [
        pltpu.VMEM((K,), i32),
        pltpu.VMEM((K,), i32),
        pltpu.VMEM((K,), i32),
        pltpu.VMEM((K,), f32),
        pltpu.VMEM((NP,), f32),
        pltpu.VMEM((NP,), f32),
        pltpu.VMEM((NP,), f32),
        pltpu.VMEM((K, D), f32),
        pltpu.VMEM((16,), i32),
        pltpu.VMEM_SHARED((NP, D), f32),
        pltpu.SemaphoreType.DMA,
    ]

    @functools.partial(
        pl.kernel, out_type=out_type, mesh=mesh, scratch_types=scratch_types,
        compiler_params=pltpu.CompilerParams(needs_layout_passes=False))
    def body(src_all, dst_all, hs_flat, att_el, att_er, feats_flat,
             segs_all, rcnt_all, zeros128,
             sum_s1, rst_f1, meanb, den_part, rden_sc, ex_sc, mean_sc,
             srcb, dstb, idxb, exh, el_t, er_t, den_t, rows, idx16,
             acc_sh, sem):
        cid = lax.axis_index("c")
        sid = lax.axis_index("s")

        def edge_sweep(chunk_fn):
            nb = (NBLKE - sid + 15) // 16

            def it(i, _):
                chunk_fn(pl.multiple_of((sid + i * 16) * K, K))
                return 0
            lax.fori_loop(0, nb, it, 0)

        def load_edges(g, base):
            eo = pl.multiple_of(g * E + base, K)
            pltpu.sync_copy(src_all.at[pl.ds(eo, K)], srcb)
            pltpu.sync_copy(dst_all.at[pl.ds(eo, K)], dstb)

        def ph1_chunk(g, h, base):
            load_edges(g, base)

            def q(j, _):
                sl = pl.ds(j * 16, 16)
                z = (plsc.load_gather(el_t, [srcb[sl]])
                     + plsc.load_gather(er_t, [dstb[sl]]))
                z = jnp.where(z >= 0.0, z, 0.2 * z)
                ex = jnp.exp(jnp.minimum(z, 60.0))
                exh[sl] = ex
                plsc.addupdate_scatter(den_t, [dstb[sl]], ex)
                return 0
            lax.fori_loop(0, K // 16, q, 0)
            pltpu.sync_copy(exh, ex_sc.at[pl.ds(pl.multiple_of(cid * E + base, K), K)])

        def ph2_chunk(g, head, base):
            load_edges(g, base)
            pltpu.sync_copy(ex_sc.at[pl.ds(pl.multiple_of(cid * E + base, K), K)], exh)

            def q(j, _):
                sl = pl.ds(j * 16, 16)
                exh[sl] = exh[sl] * plsc.load_gather(den_t, [dstb[sl]])
                idxb[sl] = srcb[sl] * H + (g * HS_ROWS + head)
                return 0
            lax.fori_loop(0, K // 16, q, 0)
            pltpu.async_copy(hs_flat.at[idxb], rows, sem).wait()

            def rowq(qq, _):
                avq = exh[pl.ds(pl.multiple_of(qq * 16, 16), 16)]
                for l in range(16):
                    av = jnp.take(avq, jnp.full((16,), l, i32))
                    k = qq * 16 + l
                    for j in range(8):
                        sl = pl.ds(j * 16, 16)
                        rows[k, sl] = rows[k, sl] * av
                return 0
            lax.fori_loop(0, K // 16, rowq, 0)
            pltpu.sync_copy(rows, acc_sh.at[dstb], add=True)

        def gat_head_pass(g, h):
            off = pl.multiple_of((g * H + h) * NP, K)
            pltpu.sync_copy(att_el.at[pl.ds(off, NP)], el_t)
            pltpu.sync_copy(att_er.at[pl.ds(off, NP)], er_t)

            def z16(i, _):
                den_t[pl.ds(i * 16, 16)] = jnp.zeros((16,), f32)
                return 0
            lax.fori_loop(0, NP // 16, z16, 0)
            edge_sweep(functools.partial(ph1_chunk, g, h))
            pltpu.sync_copy(
                den_t, den_part.at[pl.ds(pl.multiple_of((cid * 16 + sid) * NP, K), NP)])
            plsc.subcore_barrier()
            stripe = pl.multiple_of(sid * NPT, K)

            def ld(t, _):
                pltpu.sync_copy(
                    den_part.at[pl.ds(pl.multiple_of((cid * 16 + t) * NP + stripe, K), NPT)],
                    er_t.at[pl.ds(t * NPT, NPT)])
                return 0
            lax.fori_loop(0, 16, ld, 0)

            def red(qq, _):
                sl = pl.ds(qq * 16, 16)
                s = er_t[sl]
                for t in range(1, 16):
                    s = s + er_t[pl.ds(t * NPT + qq * 16, 16)]
                den_t[sl] = 1.0 / jnp.maximum(s, 1e-38)
                return 0
            lax.fori_loop(0, NPT // 16, red, 0)
            pltpu.sync_copy(den_t.at[pl.ds(0, NPT)],
                            rden_sc.at[pl.ds(pl.multiple_of(cid * NP + stripe, K), NPT)])
            plsc.subcore_barrier()
            pltpu.sync_copy(rden_sc.at[pl.ds(pl.multiple_of(cid * NP, K), NP)], den_t)
            edge_sweep(functools.partial(ph2_chunk, g, h))

        def zero_acc():
            pltpu.sync_copy(zeros128, acc_sh.at[pl.ds(sid * NPT, NPT)])
            plsc.subcore_barrier()

        def flush_acc(out_ref, head):
            plsc.subcore_barrier()
            pltpu.sync_copy(
                acc_sh.at[pl.ds(sid * NPT, NPT)],
                out_ref.at[pl.ds(pl.multiple_of(head * NP + sid * NPT, K), NPT)])
            plsc.subcore_barrier()

        def head_pass(hh, _):
            head = cid * 4 + hh
            zero_acc()

            def g_sweep(g, _):
                gat_head_pass(g, head)
                return 0
            lax.fori_loop(0, 3, g_sweep, 0)
            flush_acc(sum_s1, head)
            zero_acc()
            gat_head_pass(3, head)
            flush_acc(rst_f1, head)
            return 0

        lax.fori_loop(0, 4, head_pass, 0)

        def seg_scatter(f, base, n):
            pltpu.sync_copy(feats_flat.at[pl.ds(pl.multiple_of(f * NS + base, 8), n)],
                            rows.at[pl.ds(0, n)])
            if n == K:
                pltpu.sync_copy(
                    segs_all.at[pl.ds(pl.multiple_of(f * NP + base, K), K)], srcb)
                pltpu.sync_copy(rows, acc_sh.at[srcb], add=True)
            else:
                pltpu.sync_copy(
                    segs_all.at[pl.ds(pl.multiple_of(f * NP + NBLKN * K, K), K)], srcb)
                idx16[...] = srcb[pl.ds(0, n)]
                pltpu.sync_copy(rows.at[pl.ds(0, n)], acc_sh.at[idx16], add=True)

        def seg_bcast(f, base, n):
            if n == K:
                pltpu.sync_copy(
                    segs_all.at[pl.ds(pl.multiple_of(f * NP + base, K), K)], srcb)

                def qi(j, _):
                    sl = pl.ds(j * 16, 16)
                    idxb[sl] = srcb[sl] + f * B
                    return 0
                lax.fori_loop(0, K // 16, qi, 0)
                pltpu.async_copy(mean_sc.at[idxb], rows, sem).wait()
                pltpu.sync_copy(
                    rows, meanb.at[pl.ds(pl.multiple_of(f * NS + base, 8), n)])
            else:
                pltpu.sync_copy(
                    segs_all.at[pl.ds(pl.multiple_of(f * NP + NBLKN * K, K), K)], srcb)
                idx16[...] = srcb[pl.ds(0, n)] + f * B
                pltpu.async_copy(mean_sc.at[idx16], rows.at[pl.ds(0, n)], sem).wait()
                pltpu.sync_copy(
                    rows.at[pl.ds(0, n)],
                    meanb.at[pl.ds(pl.multiple_of(f * NS + base, 8), n)])

        def node_sweep(fn):
            nb = (NBLKN - sid + 15) // 16

            def it(i, _):
                fn(pl.multiple_of((sid + i * 16) * K, K), K)
                return 0
            lax.fori_loop(0, nb, it, 0)

            @pl.when(sid == 0)
            def _():
                fn(NBLKN * K, NTAIL)

        def seg_mean(f, _):
            fcore = jnp.where(f == 0, 0, 1)

            @pl.when(cid == fcore)
            def _():
                pltpu.sync_copy(zeros128.at[pl.ds(0, 32)],
                                acc_sh.at[pl.ds(sid * 32, 32)])
                plsc.subcore_barrier()
                node_sweep(functools.partial(seg_scatter, f))
                plsc.subcore_barrier()
                pltpu.sync_copy(acc_sh.at[pl.ds(sid * 32, 32)],
                                rows.at[pl.ds(0, 32)])
                pltpu.sync_copy(
                    rcnt_all.at[pl.ds(pl.multiple_of(f * B + sid * 32, 8), 32)],
                    rows.at[pl.ds(32, 32)])

                def mrow(r, _):
                    for j in range(8):
                        sl = pl.ds(j * 16, 16)
                        rows[r, sl] = rows[r, sl] * rows[32 + r, sl]
                    return 0
                lax.fori_loop(0, 32, mrow, 0)
                pltpu.sync_copy(
                    rows.at[pl.ds(0, 32)],
                    mean_sc.at[pl.ds(pl.multiple_of(f * B + sid * 32, 8), 32)])
                plsc.subcore_barrier()
                node_sweep(functools.partial(seg_bcast, f))
            return 0

        lax.fori_loop(0, 3, seg_mean, 0)

    return body


def kernel(feat_s1, feat_f1, feat_c1, edge_intra, edge_inter, seg_s1, seg_f1, seg_c1, params):
    f32 = jnp.float32
    ps = [params["c1_intra"], params["c2_intra"], params["c2_inter"], params["c1_inter"]]
    xs_list = [feat_s1, feat_s1, feat_f1, feat_s1]
    xd_list = [feat_s1, feat_s1, feat_s1, feat_f1]

    hs_list, el_list, er_list = [], [], []
    for p, xs, xd in zip(ps, xs_list, xd_list):
        W3 = p["W"].reshape(D, H, D)
        Wl = jnp.einsum("khd,hd->kh", W3, p["al"])
        Wr = jnp.einsum("khd,hd->kh", W3, p["ar"])
        hs = (xs @ p["W"]).reshape(NS * H, D)
        hs_list.append(hs)
        el_list.append(jnp.pad(Wl.T @ xs.T, ((0, 0), (0, NP - NS))).reshape(-1))
        er_list.append(jnp.pad(Wr.T @ xd.T, ((0, 0), (0, NP - NS))).reshape(-1))

    hs_flat = jnp.concatenate(hs_list, axis=0)
    att_el = jnp.concatenate(el_list)
    att_er = jnp.concatenate(er_list)

    ei0, ei1 = edge_intra[0], edge_intra[1]
    eI0, eI1 = edge_inter[0], edge_inter[1]
    src_all = jnp.concatenate([ei0, ei1, eI1, eI0])
    dst_all = jnp.concatenate([ei1, ei0, eI0, eI1])

    feats_flat = jnp.concatenate([feat_s1, feat_f1, feat_c1], axis=0)
    segs = [seg_s1, seg_f1, seg_c1]
    segs_all = jnp.concatenate([jnp.pad(sg, (0, NP - NS)) for sg in segs])
    rcnt_all = jnp.concatenate([
        jnp.tile((1.0 / jnp.maximum(
            jnp.zeros((B,), f32).at[sg].add(1.0), 1.0))[:, None], (1, D))
        for sg in segs])
    zeros128 = jnp.zeros((NPT, D), f32)

    sum_s1, rst_f1, meanb, _, _, _, _ = _sc_edge_kernel()(
        src_all, dst_all, hs_flat, att_el, att_er, feats_flat,
        segs_all, rcnt_all, zeros128)

    rW_s1 = ps[0]["rW"] + ps[1]["rW"] + ps[2]["rW"]
    b_s1 = ps[0]["b"] + ps[1]["b"] + ps[2]["b"]
    resid_s1 = (feat_s1 @ rW_s1 + b_s1).reshape(NS, H, D)
    resid_f1 = (feat_f1 @ ps[3]["rW"] + ps[3]["b"]).reshape(NS, H, D)

    sum_s1 = sum_s1.reshape(H, NP, D)[:, :NS]
    rst_f1 = rst_f1.reshape(H, NP, D)[:, :NS]
    h_s1 = jnp.max(sum_s1 + resid_s1.transpose(1, 0, 2), axis=0) + meanb[:NS]
    h_f1 = jnp.max(rst_f1 + resid_f1.transpose(1, 0, 2), axis=0) + meanb[NS:2 * NS]
    h_c1 = meanb[2 * NS:]
    return h_f1, h_c1, h_s1


# speedup vs baseline: 10.6428x; 1.0133x over previous
import functools

import jax
import jax.numpy as jnp
from jax import lax
from jax.experimental import pallas as pl
from jax.experimental.pallas import tpu as pltpu
from jax.experimental.pallas import tpu_sc as plsc

H = 8
D = 128
B = 512
NS = 10000
E = 160000
NG = 4
K = 128
NBLKE = E // K
NP = 10240
NPT = NP // 16
NBLKN = NS // K
NTAIL = NS - NBLKN * K
HS_ROWS = NS * H


def _sc_edge_kernel():
    mesh = plsc.VectorSubcoreMesh(core_axis_name="c", subcore_axis_name="s")
    f32 = jnp.float32
    i32 = jnp.int32

    out_type = [
        jax.ShapeDtypeStruct((H * NP, D), f32),
        jax.ShapeDtypeStruct((H * NP, D), f32),
        jax.ShapeDtypeStruct((3 * NS, D), f32),
        jax.ShapeDtypeStruct((2 * 16 * NP,), f32),
        jax.ShapeDtypeStruct((2 * NP,), f32),
        jax.ShapeDtypeStruct((2 * E,), f32),
        jax.ShapeDtypeStruct((3 * B, D), f32),
    ]
    scratch_types = [
        pltpu.VMEM((K,), i32),
        pltpu.VMEM((K,), i32),
        pltpu.VMEM((K,), i32),
        pltpu.VMEM((K,), f32),
        pltpu.VMEM((NP,), f32),
        pltpu.VMEM((NP,), f32),
        pltpu.VMEM((NP,), f32),
        pltpu.VMEM((K // 2, D), f32),
        pltpu.VMEM((K // 2, D), f32),
        pltpu.VMEM((K // 2,), i32),
        pltpu.VMEM((K // 2,), i32),
        pltpu.VMEM((K // 2,), i32),
        pltpu.VMEM((K // 2,), i32),
        pltpu.VMEM((16,), i32),
        pltpu.VMEM_SHARED((NP, D), f32),
        pltpu.SemaphoreType.DMA,
        pltpu.SemaphoreType.DMA,
    ]

    @functools.partial(
        pl.kernel, out_type=out_type, mesh=mesh, scratch_types=scratch_types,
        compiler_params=pltpu.CompilerParams(needs_layout_passes=False))
    def body(src_all, dst_all, hs_flat, att_el, att_er, feats_flat,
             segs_all, rcnt_all, zeros128,
             sum_s1, rst_f1, meanb, den_part, rden_sc, ex_sc, mean_sc,
             srcb, dstb, idxb, exh, el_t, er_t, den_t, rowsA, rowsB,
             idxA, idxB, dstA, dstB, idx16,
             acc_sh, sem, sem2):
        cid = lax.axis_index("c")
        sid = lax.axis_index("s")

        def edge_sweep(chunk_fn):
            nb = (NBLKE - sid + 15) // 16

            def it(i, _):
                chunk_fn(pl.multiple_of((sid + i * 16) * K, K))
                return 0
            lax.fori_loop(0, nb, it, 0)

        def load_edges(g, base):
            eo = pl.multiple_of(g * E + base, K)
            pltpu.sync_copy(src_all.at[pl.ds(eo, K)], srcb)
            pltpu.sync_copy(dst_all.at[pl.ds(eo, K)], dstb)

        def ph1_chunk(g, h, base):
            load_edges(g, base)

            def q(j, _):
                sl = pl.ds(j * 16, 16)
                z = (plsc.load_gather(el_t, [srcb[sl]])
                     + plsc.load_gather(er_t, [dstb[sl]]))
                z = jnp.where(z >= 0.0, z, 0.2 * z)
                ex = jnp.exp(jnp.minimum(z, 60.0))
                exh[sl] = ex
                plsc.addupdate_scatter(den_t, [dstb[sl]], ex)
                return 0
            lax.fori_loop(0, K // 16, q, 0)
            pltpu.sync_copy(exh, ex_sc.at[pl.ds(pl.multiple_of(cid * E + base, K), K)])

        def ph2_chunk(g, head, base):
            load_edges(g, base)
            pltpu.sync_copy(ex_sc.at[pl.ds(pl.multiple_of(cid * E + base, K), K)], exh)

            for j in range(4):
                sl = pl.ds(j * 16, 16)
                slb = pl.ds((j + 4) * 16, 16)
                idxA[sl] = srcb[sl] * H + (g * HS_ROWS + head)
                idxB[sl] = srcb[slb] * H + (g * HS_ROWS + head)
                dstA[sl] = dstb[sl]
                dstB[sl] = dstb[slb]
            cpA = pltpu.async_copy(hs_flat.at[idxA], rowsA, sem)
            cpB = pltpu.async_copy(hs_flat.at[idxB], rowsB, sem2)

            def q(j, _):
                sl = pl.ds(j * 16, 16)
                exh[sl] = exh[sl] * plsc.load_gather(den_t, [dstb[sl]])
                return 0
            lax.fori_loop(0, K // 16, q, 0)

            def scale(buf, eoff):
                def rowq(qq, _):
                    avq = exh[pl.ds(pl.multiple_of(eoff + qq * 16, 16), 16)]
                    for l in range(16):
                        av = jnp.take(avq, jnp.full((16,), l, i32))
                        k = qq * 16 + l
                        for j in range(8):
                            sl = pl.ds(j * 16, 16)
                            buf[k, sl] = buf[k, sl] * av
                    return 0
                lax.fori_loop(0, K // 32, rowq, 0)

            cpA.wait()
            scale(rowsA, 0)
            pltpu.sync_copy(rowsA, acc_sh.at[dstA], add=True)
            cpB.wait()
            scale(rowsB, K // 2)
            pltpu.sync_copy(rowsB, acc_sh.at[dstB], add=True)

        def gat_head_pass(g, h):
            off = pl.multiple_of((g * H + h) * NP, K)
            pltpu.sync_copy(att_el.at[pl.ds(off, NP)], el_t)
            pltpu.sync_copy(att_er.at[pl.ds(off, NP)], er_t)

            def z16(i, _):
                den_t[pl.ds(i * 16, 16)] = jnp.zeros((16,), f32)
                return 0
            lax.fori_loop(0, NP // 16, z16, 0)
            edge_sweep(functools.partial(ph1_chunk, g, h))
            pltpu.sync_copy(
                den_t, den_part.at[pl.ds(pl.multiple_of((cid * 16 + sid) * NP, K), NP)])
            plsc.subcore_barrier()
            stripe = pl.multiple_of(sid * NPT, K)

            def ld(t, _):
                pltpu.sync_copy(
                    den_part.at[pl.ds(pl.multiple_of((cid * 16 + t) * NP + stripe, K), NPT)],
                    er_t.at[pl.ds(t * NPT, NPT)])
                return 0
            lax.fori_loop(0, 16, ld, 0)

            def red(qq, _):
                sl = pl.ds(qq * 16, 16)
                s = er_t[sl]
                for t in range(1, 16):
                    s = s + er_t[pl.ds(t * NPT + qq * 16, 16)]
                den_t[sl] = 1.0 / jnp.maximum(s, 1e-38)
                return 0
            lax.fori_loop(0, NPT // 16, red, 0)
            pltpu.sync_copy(den_t.at[pl.ds(0, NPT)],
                            rden_sc.at[pl.ds(pl.multiple_of(cid * NP + stripe, K), NPT)])
            plsc.subcore_barrier()
            pltpu.sync_copy(rden_sc.at[pl.ds(pl.multiple_of(cid * NP, K), NP)], den_t)
            edge_sweep(functools.partial(ph2_chunk, g, h))

        def zero_acc():
            pltpu.sync_copy(zeros128, acc_sh.at[pl.ds(sid * NPT, NPT)])
            plsc.subcore_barrier()

        def flush_acc(out_ref, head):
            plsc.subcore_barrier()
            pltpu.sync_copy(
                acc_sh.at[pl.ds(sid * NPT, NPT)],
                out_ref.at[pl.ds(pl.multiple_of(head * NP + sid * NPT, K), NPT)])
            plsc.subcore_barrier()

        def head_pass(hh, _):
            head = cid * 4 + hh
            zero_acc()

            def g_sweep(g, _):
                gat_head_pass(g, head)
                return 0
            lax.fori_loop(0, 3, g_sweep, 0)
            flush_acc(sum_s1, head)
            zero_acc()
            gat_head_pass(3, head)
            flush_acc(rst_f1, head)
            return 0

        lax.fori_loop(0, 4, head_pass, 0)

        def seg_scatter(f, base, n):
            if n == K:
                fo = pl.multiple_of(f * NS + base, 8)
                pltpu.sync_copy(feats_flat.at[pl.ds(fo, K // 2)], rowsA)
                pltpu.sync_copy(feats_flat.at[pl.ds(pl.multiple_of(fo + K // 2, 8), K // 2)],
                                rowsB)
                pltpu.sync_copy(
                    segs_all.at[pl.ds(pl.multiple_of(f * NP + base, K), K)], srcb)
                for j in range(4):
                    sl = pl.ds(j * 16, 16)
                    dstA[sl] = srcb[sl]
                    dstB[sl] = srcb[pl.ds((j + 4) * 16, 16)]
                pltpu.sync_copy(rowsA, acc_sh.at[dstA], add=True)
                pltpu.sync_copy(rowsB, acc_sh.at[dstB], add=True)
            else:
                pltpu.sync_copy(feats_flat.at[pl.ds(pl.multiple_of(f * NS + base, 8), n)],
                                rowsA.at[pl.ds(0, n)])
                pltpu.sync_copy(
                    segs_all.at[pl.ds(pl.multiple_of(f * NP + NBLKN * K, K), K)], srcb)
                idx16[...] = srcb[pl.ds(0, n)]
                pltpu.sync_copy(rowsA.at[pl.ds(0, n)], acc_sh.at[idx16], add=True)

        def seg_bcast(f, base, n):
            if n == K:
                pltpu.sync_copy(
                    segs_all.at[pl.ds(pl.multiple_of(f * NP + base, K), K)], srcb)
                for j in range(4):
                    sl = pl.ds(j * 16, 16)
                    dstA[sl] = srcb[sl] + f * B
                    dstB[sl] = srcb[pl.ds((j + 4) * 16, 16)] + f * B
                cpA = pltpu.async_copy(mean_sc.at[dstA], rowsA, sem)
                cpB = pltpu.async_copy(mean_sc.at[dstB], rowsB, sem2)
                cpA.wait()
                cpB.wait()
                fo = pl.multiple_of(f * NS + base, 8)
                pltpu.sync_copy(rowsA, meanb.at[pl.ds(fo, K // 2)])
                pltpu.sync_copy(rowsB,
                                meanb.at[pl.ds(pl.multiple_of(fo + K // 2, 8), K // 2)])
            else:
                pltpu.sync_copy(
                    segs_all.at[pl.ds(pl.multiple_of(f * NP + NBLKN * K, K), K)], srcb)
                idx16[...] = srcb[pl.ds(0, n)] + f * B
                pltpu.async_copy(mean_sc.at[idx16], rowsA.at[pl.ds(0, n)], sem).wait()
                pltpu.sync_copy(
                    rowsA.at[pl.ds(0, n)],
                    meanb.at[pl.ds(pl.multiple_of(f * NS + base, 8), n)])

        def node_sweep(fn):
            nb = (NBLKN - sid + 15) // 16

            def it(i, _):
                fn(pl.multiple_of((sid + i * 16) * K, K), K)
                return 0
            lax.fori_loop(0, nb, it, 0)

            @pl.when(sid == 0)
            def _():
                fn(NBLKN * K, NTAIL)

        def seg_mean(f, _):
            fcore = jnp.where(f == 0, 0, 1)

            @pl.when(cid == fcore)
            def _():
                pltpu.sync_copy(zeros128.at[pl.ds(0, 32)],
                                acc_sh.at[pl.ds(sid * 32, 32)])
                plsc.subcore_barrier()
                node_sweep(functools.partial(seg_scatter, f))
                plsc.subcore_barrier()
                pltpu.sync_copy(acc_sh.at[pl.ds(sid * 32, 32)],
                                rowsA.at[pl.ds(0, 32)])
                pltpu.sync_copy(
                    rcnt_all.at[pl.ds(pl.multiple_of(f * B + sid * 32, 8), 32)],
                    rowsA.at[pl.ds(32, 32)])

                def mrow(r, _):
                    for j in range(8):
                        sl = pl.ds(j * 16, 16)
                        rowsA[r, sl] = rowsA[r, sl] * rowsA[32 + r, sl]
                    return 0
                lax.fori_loop(0, 32, mrow, 0)
                pltpu.sync_copy(
                    rowsA.at[pl.ds(0, 32)],
                    mean_sc.at[pl.ds(pl.multiple_of(f * B + sid * 32, 8), 32)])
                plsc.subcore_barrier()
                node_sweep(functools.partial(seg_bcast, f))
            return 0

        lax.fori_loop(0, 3, seg_mean, 0)

    return body


def kernel(feat_s1, feat_f1, feat_c1, edge_intra, edge_inter, seg_s1, seg_f1, seg_c1, params):
    f32 = jnp.float32
    ps = [params["c1_intra"], params["c2_intra"], params["c2_inter"], params["c1_inter"]]
    xs_list = [feat_s1, feat_s1, feat_f1, feat_s1]
    xd_list = [feat_s1, feat_s1, feat_s1, feat_f1]

    hs_list, el_list, er_list = [], [], []
    for p, xs, xd in zip(ps, xs_list, xd_list):
        W3 = p["W"].reshape(D, H, D)
        Wl = jnp.einsum("khd,hd->kh", W3, p["al"])
        Wr = jnp.einsum("khd,hd->kh", W3, p["ar"])
        hs = (xs @ p["W"]).reshape(NS * H, D)
        hs_list.append(hs)
        el_list.append(jnp.pad(Wl.T @ xs.T, ((0, 0), (0, NP - NS))).reshape(-1))
        er_list.append(jnp.pad(Wr.T @ xd.T, ((0, 0), (0, NP - NS))).reshape(-1))

    hs_flat = jnp.concatenate(hs_list, axis=0)
    att_el = jnp.concatenate(el_list)
    att_er = jnp.concatenate(er_list)

    ei0, ei1 = edge_intra[0], edge_intra[1]
    eI0, eI1 = edge_inter[0], edge_inter[1]
    src_all = jnp.concatenate([ei0, ei1, eI1, eI0])
    dst_all = jnp.concatenate([ei1, ei0, eI0, eI1])

    feats_flat = jnp.concatenate([feat_s1, feat_f1, feat_c1], axis=0)
    segs = [seg_s1, seg_f1, seg_c1]
    segs_all = jnp.concatenate([jnp.pad(sg, (0, NP - NS)) for sg in segs])
    rcnt_all = jnp.concatenate([
        jnp.tile((1.0 / jnp.maximum(
            jnp.zeros((B,), f32).at[sg].add(1.0), 1.0))[:, None], (1, D))
        for sg in segs])
    zeros128 = jnp.zeros((NPT, D), f32)

    sum_s1, rst_f1, meanb, _, _, _, _ = _sc_edge_kernel()(
        src_all, dst_all, hs_flat, att_el, att_er, feats_flat,
        segs_all, rcnt_all, zeros128)

    rW_s1 = ps[0]["rW"] + ps[1]["rW"] + ps[2]["rW"]
    b_s1 = ps[0]["b"] + ps[1]["b"] + ps[2]["b"]
    resid_s1 = (feat_s1 @ rW_s1 + b_s1).reshape(NS, H, D)
    resid_f1 = (feat_f1 @ ps[3]["rW"] + ps[3]["b"]).reshape(NS, H, D)

    sum_s1 = sum_s1.reshape(H, NP, D)[:, :NS]
    rst_f1 = rst_f1.reshape(H, NP, D)[:, :NS]
    h_s1 = jnp.max(sum_s1 + resid_s1.transpose(1, 0, 2), axis=0) + meanb[:NS]
    h_f1 = jnp.max(rst_f1 + resid_f1.transpose(1, 0, 2), axis=0) + meanb[NS:2 * NS]
    h_c1 = meanb[2 * NS:]
    return h_f1, h_c1, h_s1


# speedup vs baseline: 15.0838x; 1.4173x over previous
import functools

import jax
import jax.numpy as jnp
from jax import lax
from jax.experimental import pallas as pl
from jax.experimental.pallas import tpu as pltpu
from jax.experimental.pallas import tpu_sc as plsc

H = 8
D = 128
B = 512
NS = 10000
E = 160000
NG = 4
K = 128
NBLKE = E // K
NP = 10240
NPT = NP // 16
NBLKN = NS // K
NTAIL = NS - NBLKN * K
HS_ROWS = NS * H


def _sc_edge_kernel():
    mesh = plsc.VectorSubcoreMesh(core_axis_name="c", subcore_axis_name="s")
    f32 = jnp.float32
    i32 = jnp.int32

    out_type = [
        jax.ShapeDtypeStruct((H * NP, D), f32),
        jax.ShapeDtypeStruct((H * NP, D), f32),
        jax.ShapeDtypeStruct((3 * NS, D), f32),
        jax.ShapeDtypeStruct((2 * 16 * NP,), f32),
        jax.ShapeDtypeStruct((2 * NP,), f32),
        jax.ShapeDtypeStruct((2 * E,), f32),
        jax.ShapeDtypeStruct((3 * B, D), f32),
    ]
    scratch_types = [
        pltpu.VMEM((K,), i32),
        pltpu.VMEM((K,), i32),
        pltpu.VMEM((K,), i32),
        pltpu.VMEM((K,), f32),
        pltpu.VMEM((NP,), f32),
        pltpu.VMEM((NP,), f32),
        pltpu.VMEM((NP,), f32),
        pltpu.VMEM((K // 2, D), f32),
        pltpu.VMEM((K // 2, D), f32),
        pltpu.VMEM((K // 2,), i32),
        pltpu.VMEM((K // 2,), i32),
        pltpu.VMEM((K // 2,), i32),
        pltpu.VMEM((K // 2,), i32),
        pltpu.VMEM((16,), i32),
        pltpu.VMEM((2 * K,), i32),
        pltpu.VMEM((2 * K,), i32),
        pltpu.VMEM((K,), f32),
        pltpu.VMEM_SHARED((NP, D), f32),
        pltpu.SemaphoreType.DMA,
        pltpu.SemaphoreType.DMA,
        pltpu.SemaphoreType.DMA,
        pltpu.SemaphoreType.DMA,
    ]

    @functools.partial(
        pl.kernel, out_type=out_type, mesh=mesh, scratch_types=scratch_types,
        compiler_params=pltpu.CompilerParams(needs_layout_passes=False))
    def body(edg_all, hs_flat, att_el, att_er, feats_flat,
             segs_all, rcnt_all, zeros128,
             sum_s1, rst_f1, meanb, den_part, rden_sc, ex_sc, mean_sc,
             srcb, dstb, idxb, exh, el_t, er_t, den_t, rowsA, rowsB,
             idxA, idxB, dstA, dstB, idx16, edg0, edg1, exh1,
             acc_sh, sem, sem2, semP0, semP1):
        cid = lax.axis_index("c")
        sid = lax.axis_index("s")

        def edge_sweep(chunk_fn):
            nb = (NBLKE - sid + 15) // 16

            def it(i, _):
                chunk_fn(pl.multiple_of((sid + i * 16) * K, K))
                return 0
            lax.fori_loop(0, nb, it, 0)

        def ph1_chunk(g, h, base):
            eo = pl.multiple_of(2 * g * E + 2 * base, 2 * K)
            pltpu.sync_copy(edg_all.at[pl.ds(eo, 2 * K)], edg0)

            def q(j, _):
                sl = pl.ds(j * 16, 16)
                sld = pl.ds(K + j * 16, 16)
                dv = edg0[sld]
                z = (plsc.load_gather(el_t, [edg0[sl]])
                     + plsc.load_gather(er_t, [dv]))
                z = jnp.where(z >= 0.0, z, 0.2 * z)
                ex = jnp.exp(jnp.minimum(z, 60.0))
                exh[sl] = ex
                plsc.addupdate_scatter(den_t, [dv], ex)
                return 0
            lax.fori_loop(0, K // 16, q, 0)
            pltpu.sync_copy(exh, ex_sc.at[pl.ds(pl.multiple_of(cid * E + base, K), K)])

        def ph2_prefetch(g, blk, edgb, exhb, semP):
            eo = pl.multiple_of(2 * g * E + blk * (2 * K), 2 * K)
            xo = pl.multiple_of(cid * E + blk * K, K)
            c1 = pltpu.async_copy(edg_all.at[pl.ds(eo, 2 * K)], edgb, semP)
            c2 = pltpu.async_copy(ex_sc.at[pl.ds(xo, K)], exhb, semP)
            return c1, c2

        def ph2_process(g, head, edgb, exhb):
            for j in range(4):
                sl = pl.ds(j * 16, 16)
                slb = pl.ds((j + 4) * 16, 16)
                idxA[sl] = edgb[sl] * H + (g * HS_ROWS + head)
                idxB[sl] = edgb[slb] * H + (g * HS_ROWS + head)
                dstA[sl] = edgb[pl.ds(K + j * 16, 16)]
                dstB[sl] = edgb[pl.ds(K + (j + 4) * 16, 16)]
            cpA = pltpu.async_copy(hs_flat.at[idxA], rowsA, sem)
            cpB = pltpu.async_copy(hs_flat.at[idxB], rowsB, sem2)

            def q(j, _):
                sl = pl.ds(j * 16, 16)
                exhb[sl] = exhb[sl] * plsc.load_gather(
                    den_t, [edgb[pl.ds(K + j * 16, 16)]])
                return 0
            lax.fori_loop(0, K // 16, q, 0)

            def scale(buf, eoff):
                def rowq(qq, _):
                    avq = exhb[pl.ds(pl.multiple_of(eoff + qq * 16, 16), 16)]
                    for l in range(16):
                        av = jnp.take(avq, jnp.full((16,), l, i32))
                        k = qq * 16 + l
                        for j in range(8):
                            sl = pl.ds(j * 16, 16)
                            buf[k, sl] = buf[k, sl] * av
                    return 0
                lax.fori_loop(0, K // 32, rowq, 0)

            cpA.wait()
            scale(rowsA, 0)
            pltpu.sync_copy(rowsA, acc_sh.at[dstA], add=True)
            cpB.wait()
            scale(rowsB, K // 2)
            pltpu.sync_copy(rowsB, acc_sh.at[dstB], add=True)

        def ph2_sweep(g, head):
            nb = (NBLKE - sid + 15) // 16
            pf0 = ph2_prefetch(g, sid, edg0, exh, semP0)

            def pair(m, _):
                i0 = 2 * m

                @pl.when(i0 + 1 < nb)
                def _():
                    ph2_prefetch(g, sid + (i0 + 1) * 16, edg1, exh1, semP1)
                pltpu.make_async_copy(edg_all.at[pl.ds(0, 2 * K)], edg0, semP0).wait()
                pltpu.make_async_copy(ex_sc.at[pl.ds(0, K)], exh, semP0).wait()
                ph2_process(g, head, edg0, exh)

                @pl.when(i0 + 1 < nb)
                def _():
                    @pl.when(i0 + 2 < nb)
                    def _():
                        ph2_prefetch(g, sid + (i0 + 2) * 16, edg0, exh, semP0)
                    pltpu.make_async_copy(edg_all.at[pl.ds(0, 2 * K)], edg1, semP1).wait()
                    pltpu.make_async_copy(ex_sc.at[pl.ds(0, K)], exh1, semP1).wait()
                    ph2_process(g, head, edg1, exh1)
                return 0
            lax.fori_loop(0, (nb + 1) // 2, pair, 0)

        def gat_head_pass(g, h):
            off = pl.multiple_of((g * H + h) * NP, K)
            pltpu.sync_copy(att_el.at[pl.ds(off, NP)], el_t)
            pltpu.sync_copy(att_er.at[pl.ds(off, NP)], er_t)

            def z16(i, _):
                den_t[pl.ds(i * 16, 16)] = jnp.zeros((16,), f32)
                return 0
            lax.fori_loop(0, NP // 16, z16, 0)
            edge_sweep(functools.partial(ph1_chunk, g, h))
            pltpu.sync_copy(
                den_t, den_part.at[pl.ds(pl.multiple_of((cid * 16 + sid) * NP, K), NP)])
            plsc.subcore_barrier()
            stripe = pl.multiple_of(sid * NPT, K)

            def ld(t, _):
                pltpu.sync_copy(
                    den_part.at[pl.ds(pl.multiple_of((cid * 16 + t) * NP + stripe, K), NPT)],
                    er_t.at[pl.ds(t * NPT, NPT)])
                return 0
            lax.fori_loop(0, 16, ld, 0)

            def red(qq, _):
                sl = pl.ds(qq * 16, 16)
                s = er_t[sl]
                for t in range(1, 16):
                    s = s + er_t[pl.ds(t * NPT + qq * 16, 16)]
                den_t[sl] = 1.0 / jnp.maximum(s, 1e-38)
                return 0
            lax.fori_loop(0, NPT // 16, red, 0)
            pltpu.sync_copy(den_t.at[pl.ds(0, NPT)],
                            rden_sc.at[pl.ds(pl.multiple_of(cid * NP + stripe, K), NPT)])
            plsc.subcore_barrier()
            pltpu.sync_copy(rden_sc.at[pl.ds(pl.multiple_of(cid * NP, K), NP)], den_t)
            ph2_sweep(g, h)

        def zero_acc():
            pltpu.sync_copy(zeros128, acc_sh.at[pl.ds(sid * NPT, NPT)])
            plsc.subcore_barrier()

        def flush_acc(out_ref, head):
            plsc.subcore_barrier()
            pltpu.sync_copy(
                acc_sh.at[pl.ds(sid * NPT, NPT)],
                out_ref.at[pl.ds(pl.multiple_of(head * NP + sid * NPT, K), NPT)])
            plsc.subcore_barrier()

        def head_pass(hh, _):
            head = cid * 4 + hh
            zero_acc()

            def g_sweep(g, _):
                gat_head_pass(g, head)
                return 0
            lax.fori_loop(0, 3, g_sweep, 0)
            flush_acc(sum_s1, head)
            zero_acc()
            gat_head_pass(3, head)
            flush_acc(rst_f1, head)
            return 0

        lax.fori_loop(0, 4, head_pass, 0)

        def seg_scatter(f, base, n):
            if n == K:
                fo = pl.multiple_of(f * NS + base, 8)
                pltpu.sync_copy(feats_flat.at[pl.ds(fo, K // 2)], rowsA)
                pltpu.sync_copy(feats_flat.at[pl.ds(pl.multiple_of(fo + K // 2, 8), K // 2)],
                                rowsB)
                pltpu.sync_copy(
                    segs_all.at[pl.ds(pl.multiple_of(f * NP + base, K), K)], srcb)
                for j in range(4):
                    sl = pl.ds(j * 16, 16)
                    dstA[sl] = srcb[sl]
                    dstB[sl] = srcb[pl.ds((j + 4) * 16, 16)]
                pltpu.sync_copy(rowsA, acc_sh.at[dstA], add=True)
                pltpu.sync_copy(rowsB, acc_sh.at[dstB], add=True)
            else:
                pltpu.sync_copy(feats_flat.at[pl.ds(pl.multiple_of(f * NS + base, 8), n)],
                                rowsA.at[pl.ds(0, n)])
                pltpu.sync_copy(
                    segs_all.at[pl.ds(pl.multiple_of(f * NP + NBLKN * K, K), K)], srcb)
                idx16[...] = srcb[pl.ds(0, n)]
                pltpu.sync_copy(rowsA.at[pl.ds(0, n)], acc_sh.at[idx16], add=True)

        def seg_bcast(f, base, n):
            if n == K:
                pltpu.sync_copy(
                    segs_all.at[pl.ds(pl.multiple_of(f * NP + base, K), K)], srcb)
                for j in range(4):
                    sl = pl.ds(j * 16, 16)
                    dstA[sl] = srcb[sl] + f * B
                    dstB[sl] = srcb[pl.ds((j + 4) * 16, 16)] + f * B
                cpA = pltpu.async_copy(mean_sc.at[dstA], rowsA, sem)
                cpB = pltpu.async_copy(mean_sc.at[dstB], rowsB, sem2)
                cpA.wait()
                cpB.wait()
                fo = pl.multiple_of(f * NS + base, 8)
                pltpu.sync_copy(rowsA, meanb.at[pl.ds(fo, K // 2)])
                pltpu.sync_copy(rowsB,
                                meanb.at[pl.ds(pl.multiple_of(fo + K // 2, 8), K // 2)])
            else:
                pltpu.sync_copy(
                    segs_all.at[pl.ds(pl.multiple_of(f * NP + NBLKN * K, K), K)], srcb)
                idx16[...] = srcb[pl.ds(0, n)] + f * B
                pltpu.async_copy(mean_sc.at[idx16], rowsA.at[pl.ds(0, n)], sem).wait()
                pltpu.sync_copy(
                    rowsA.at[pl.ds(0, n)],
                    meanb.at[pl.ds(pl.multiple_of(f * NS + base, 8), n)])

        def node_sweep(fn):
            nb = (NBLKN - sid + 15) // 16

            def it(i, _):
                fn(pl.multiple_of((sid + i * 16) * K, K), K)
                return 0
            lax.fori_loop(0, nb, it, 0)

            @pl.when(sid == 0)
            def _():
                fn(NBLKN * K, NTAIL)

        def seg_mean(f, _):
            fcore = jnp.where(f == 0, 0, 1)

            @pl.when(cid == fcore)
            def _():
                pltpu.sync_copy(zeros128.at[pl.ds(0, 32)],
                                acc_sh.at[pl.ds(sid * 32, 32)])
                plsc.subcore_barrier()
                node_sweep(functools.partial(seg_scatter, f))
                plsc.subcore_barrier()
                pltpu.sync_copy(acc_sh.at[pl.ds(sid * 32, 32)],
                                rowsA.at[pl.ds(0, 32)])
                pltpu.sync_copy(
                    rcnt_all.at[pl.ds(pl.multiple_of(f * B + sid * 32, 8), 32)],
                    rowsA.at[pl.ds(32, 32)])

                def mrow(r, _):
                    for j in range(8):
                        sl = pl.ds(j * 16, 16)
                        rowsA[r, sl] = rowsA[r, sl] * rowsA[32 + r, sl]
                    return 0
                lax.fori_loop(0, 32, mrow, 0)
                pltpu.sync_copy(
                    rowsA.at[pl.ds(0, 32)],
                    mean_sc.at[pl.ds(pl.multiple_of(f * B + sid * 32, 8), 32)])
                plsc.subcore_barrier()
                node_sweep(functools.partial(seg_bcast, f))
            return 0

        lax.fori_loop(0, 3, seg_mean, 0)

    return body


def kernel(feat_s1, feat_f1, feat_c1, edge_intra, edge_inter, seg_s1, seg_f1, seg_c1, params):
    f32 = jnp.float32
    ps = [params["c1_intra"], params["c2_intra"], params["c2_inter"], params["c1_inter"]]
    xs_list = [feat_s1, feat_s1, feat_f1, feat_s1]
    xd_list = [feat_s1, feat_s1, feat_s1, feat_f1]

    hs_list, el_list, er_list = [], [], []
    for p, xs, xd in zip(ps, xs_list, xd_list):
        W3 = p["W"].reshape(D, H, D)
        Wl = jnp.einsum("khd,hd->kh", W3, p["al"])
        Wr = jnp.einsum("khd,hd->kh", W3, p["ar"])
        hs = (xs @ p["W"]).reshape(NS * H, D)
        hs_list.append(hs)
        el_list.append(jnp.pad(Wl.T @ xs.T, ((0, 0), (0, NP - NS))).reshape(-1))
        er_list.append(jnp.pad(Wr.T @ xd.T, ((0, 0), (0, NP - NS))).reshape(-1))

    hs_flat = jnp.concatenate(hs_list, axis=0)
    att_el = jnp.concatenate(el_list)
    att_er = jnp.concatenate(er_list)

    ei0, ei1 = edge_intra[0], edge_intra[1]
    eI0, eI1 = edge_inter[0], edge_inter[1]
    src_all = jnp.concatenate([ei0, ei1, eI1, eI0])
    dst_all = jnp.concatenate([ei1, ei0, eI0, eI1])
    edg_all = jnp.stack([src_all.reshape(NG * NBLKE, K),
                         dst_all.reshape(NG * NBLKE, K)], axis=1).reshape(-1)

    feats_flat = jnp.concatenate([feat_s1, feat_f1, feat_c1], axis=0)
    segs = [seg_s1, seg_f1, seg_c1]
    segs_all = jnp.concatenate([jnp.pad(sg, (0, NP - NS)) for sg in segs])
    rcnt_all = jnp.concatenate([
        jnp.tile((1.0 / jnp.maximum(
            jnp.zeros((B,), f32).at[sg].add(1.0), 1.0))[:, None], (1, D))
        for sg in segs])
    zeros128 = jnp.zeros((NPT, D), f32)

    sum_s1, rst_f1, meanb, _, _, _, _ = _sc_edge_kernel()(
        edg_all, hs_flat, att_el, att_er, feats_flat,
        segs_all, rcnt_all, zeros128)

    rW_s1 = ps[0]["rW"] + ps[1]["rW"] + ps[2]["rW"]
    b_s1 = ps[0]["b"] + ps[1]["b"] + ps[2]["b"]
    resid_s1 = (feat_s1 @ rW_s1 + b_s1).reshape(NS, H, D)
    resid_f1 = (feat_f1 @ ps[3]["rW"] + ps[3]["b"]).reshape(NS, H, D)

    sum_s1 = sum_s1.reshape(H, NP, D)[:, :NS]
    rst_f1 = rst_f1.reshape(H, NP, D)[:, :NS]
    h_s1 = jnp.max(sum_s1 + resid_s1.transpose(1, 0, 2), axis=0) + meanb[:NS]
    h_f1 = jnp.max(rst_f1 + resid_f1.transpose(1, 0, 2), axis=0) + meanb[NS:2 * NS]
    h_c1 = meanb[2 * NS:]
    return h_f1, h_c1, h_s1


# speedup vs baseline: 17.4964x; 1.1599x over previous
import functools

import jax
import jax.numpy as jnp
from jax import lax
from jax.experimental import pallas as pl
from jax.experimental.pallas import tpu as pltpu
from jax.experimental.pallas import tpu_sc as plsc

H = 8
D = 128
B = 512
NS = 10000
E = 160000
NG = 4
K = 128
NBLKE = E // K
NP = 10240
NPT = NP // 16
NBLKN = NS // K
NTAIL = NS - NBLKN * K
HS_ROWS = NS * H


def _sc_edge_kernel():
    mesh = plsc.VectorSubcoreMesh(core_axis_name="c", subcore_axis_name="s")
    f32 = jnp.float32
    i32 = jnp.int32

    out_type = [
        jax.ShapeDtypeStruct((H * NP, D), f32),
        jax.ShapeDtypeStruct((H * NP, D), f32),
        jax.ShapeDtypeStruct((3 * NS, D), f32),
        jax.ShapeDtypeStruct((2 * 16 * NP,), f32),
        jax.ShapeDtypeStruct((2 * NP,), f32),
        jax.ShapeDtypeStruct((2 * E,), f32),
        jax.ShapeDtypeStruct((3 * B, D), f32),
    ]
    scratch_types = [
        pltpu.VMEM((K,), i32),
        pltpu.VMEM((K,), i32),
        pltpu.VMEM((K,), i32),
        pltpu.VMEM((K,), f32),
        pltpu.VMEM((NP,), f32),
        pltpu.VMEM((NP,), f32),
        pltpu.VMEM((NP,), f32),
        pltpu.VMEM((K // 2, D), f32),
        pltpu.VMEM((K // 2, D), f32),
        pltpu.VMEM((K // 2,), i32),
        pltpu.VMEM((K // 2,), i32),
        pltpu.VMEM((K // 2,), i32),
        pltpu.VMEM((K // 2,), i32),
        pltpu.VMEM((16,), i32),
        pltpu.VMEM((2 * K,), i32),
        pltpu.VMEM((2 * K,), i32),
        pltpu.VMEM((K,), f32),
        pltpu.VMEM_SHARED((NP, D), f32),
        pltpu.SemaphoreType.DMA,
        pltpu.SemaphoreType.DMA,
        pltpu.SemaphoreType.DMA,
        pltpu.SemaphoreType.DMA,
        pltpu.SemaphoreType.DMA,
    ]

    @functools.partial(
        pl.kernel, out_type=out_type, mesh=mesh, scratch_types=scratch_types,
        compiler_params=pltpu.CompilerParams(needs_layout_passes=False))
    def body(edg_all, hs_flat, att_el, att_er, feats_flat,
             segs_all, rcnt_all, zeros128,
             sum_s1, rst_f1, meanb, den_part, rden_sc, ex_sc, mean_sc,
             srcb, dstb, idxb, exh, el_t, er_t, den_t, rowsA, rowsB,
             idxA, idxB, dstA, dstB, idx16, edg0, edg1, exh1,
             acc_sh, sem, sem2, semP0, semP1, semS):
        cid = lax.axis_index("c")
        sid = lax.axis_index("s")

        def edge_sweep(chunk_fn):
            nb = (NBLKE - sid + 15) // 16

            def it(i, _):
                chunk_fn(pl.multiple_of((sid + i * 16) * K, K))
                return 0
            lax.fori_loop(0, nb, it, 0)

        def ph1_prefetch(g, blk, edgb, semP):
            eo = pl.multiple_of(2 * g * E + blk * (2 * K), 2 * K)
            pltpu.async_copy(edg_all.at[pl.ds(eo, 2 * K)], edgb, semP)

        def ph1_process(g, base, edgb):
            def q(j, _):
                sl = pl.ds(j * 16, 16)
                dv = edgb[pl.ds(K + j * 16, 16)]
                z = (plsc.load_gather(el_t, [edgb[sl]])
                     + plsc.load_gather(er_t, [dv]))
                z = jnp.where(z >= 0.0, z, 0.2 * z)
                ex = jnp.exp(jnp.minimum(z, 60.0))
                exh[sl] = ex
                plsc.addupdate_scatter(den_t, [dv], ex)
                return 0
            lax.fori_loop(0, K // 16, q, 0)
            pltpu.sync_copy(exh, ex_sc.at[pl.ds(pl.multiple_of(cid * E + base, K), K)])

        def ph1_sweep(g, h):
            nb = (NBLKE - sid + 15) // 16
            ph1_prefetch(g, sid, edg0, semP0)

            def pair(m, _):
                i0 = 2 * m

                @pl.when(i0 + 1 < nb)
                def _():
                    ph1_prefetch(g, sid + (i0 + 1) * 16, edg1, semP1)
                pltpu.make_async_copy(edg_all.at[pl.ds(0, 2 * K)], edg0, semP0).wait()
                ph1_process(g, (sid + i0 * 16) * K, edg0)

                @pl.when(i0 + 1 < nb)
                def _():
                    @pl.when(i0 + 2 < nb)
                    def _():
                        ph1_prefetch(g, sid + (i0 + 2) * 16, edg0, semP0)
                    pltpu.make_async_copy(edg_all.at[pl.ds(0, 2 * K)], edg1, semP1).wait()
                    ph1_process(g, (sid + (i0 + 1) * 16) * K, edg1)
                return 0
            lax.fori_loop(0, (nb + 1) // 2, pair, 0)

        def ph2_prefetch(g, blk, edgb, exhb, semP):
            eo = pl.multiple_of(2 * g * E + blk * (2 * K), 2 * K)
            xo = pl.multiple_of(cid * E + blk * K, K)
            c1 = pltpu.async_copy(edg_all.at[pl.ds(eo, 2 * K)], edgb, semP)
            c2 = pltpu.async_copy(ex_sc.at[pl.ds(xo, K)], exhb, semP)
            return c1, c2

        def ph2_process(g, head, edgb, exhb):
            for j in range(4):
                sl = pl.ds(j * 16, 16)
                slb = pl.ds((j + 4) * 16, 16)
                idxA[sl] = edgb[sl] * H + (g * HS_ROWS + head)
                idxB[sl] = edgb[slb] * H + (g * HS_ROWS + head)
                dstA[sl] = edgb[pl.ds(K + j * 16, 16)]
                dstB[sl] = edgb[pl.ds(K + (j + 4) * 16, 16)]
            cpA = pltpu.async_copy(hs_flat.at[idxA], rowsA, sem)
            cpB = pltpu.async_copy(hs_flat.at[idxB], rowsB, sem2)

            def q(j, _):
                sl = pl.ds(j * 16, 16)
                exhb[sl] = exhb[sl] * plsc.load_gather(
                    den_t, [edgb[pl.ds(K + j * 16, 16)]])
                return 0
            lax.fori_loop(0, K // 16, q, 0)

            def scale(buf, eoff):
                def rowq(qq, _):
                    avq = exhb[pl.ds(pl.multiple_of(eoff + qq * 16, 16), 16)]
                    for l in range(16):
                        av = jnp.take(avq, jnp.full((16,), l, i32))
                        k = qq * 16 + l
                        for j in range(8):
                            sl = pl.ds(j * 16, 16)
                            buf[k, sl] = buf[k, sl] * av
                    return 0
                lax.fori_loop(0, K // 32, rowq, 0)

            cpA.wait()
            scale(rowsA, 0)
            scA = pltpu.async_copy(rowsA, acc_sh.at[dstA], add=True, sem=semS)
            cpB.wait()
            scale(rowsB, K // 2)
            scA.wait()
            pltpu.sync_copy(rowsB, acc_sh.at[dstB], add=True)

        def ph2_sweep(g, head):
            nb = (NBLKE - sid + 15) // 16
            pf0 = ph2_prefetch(g, sid, edg0, exh, semP0)

            def pair(m, _):
                i0 = 2 * m

                @pl.when(i0 + 1 < nb)
                def _():
                    ph2_prefetch(g, sid + (i0 + 1) * 16, edg1, exh1, semP1)
                pltpu.make_async_copy(edg_all.at[pl.ds(0, 2 * K)], edg0, semP0).wait()
                pltpu.make_async_copy(ex_sc.at[pl.ds(0, K)], exh, semP0).wait()
                ph2_process(g, head, edg0, exh)

                @pl.when(i0 + 1 < nb)
                def _():
                    @pl.when(i0 + 2 < nb)
                    def _():
                        ph2_prefetch(g, sid + (i0 + 2) * 16, edg0, exh, semP0)
                    pltpu.make_async_copy(edg_all.at[pl.ds(0, 2 * K)], edg1, semP1).wait()
                    pltpu.make_async_copy(ex_sc.at[pl.ds(0, K)], exh1, semP1).wait()
                    ph2_process(g, head, edg1, exh1)
                return 0
            lax.fori_loop(0, (nb + 1) // 2, pair, 0)

        def gat_head_pass(g, h):
            off = pl.multiple_of((g * H + h) * NP, K)
            pltpu.sync_copy(att_el.at[pl.ds(off, NP)], el_t)
            pltpu.sync_copy(att_er.at[pl.ds(off, NP)], er_t)

            def z16(i, _):
                den_t[pl.ds(i * 16, 16)] = jnp.zeros((16,), f32)
                return 0
            lax.fori_loop(0, NP // 16, z16, 0)
            ph1_sweep(g, h)
            pltpu.sync_copy(
                den_t, den_part.at[pl.ds(pl.multiple_of((cid * 16 + sid) * NP, K), NP)])
            plsc.subcore_barrier()
            stripe = pl.multiple_of(sid * NPT, K)

            def ld(t, _):
                pltpu.sync_copy(
                    den_part.at[pl.ds(pl.multiple_of((cid * 16 + t) * NP + stripe, K), NPT)],
                    er_t.at[pl.ds(t * NPT, NPT)])
                return 0
            lax.fori_loop(0, 16, ld, 0)

            def red(qq, _):
                sl = pl.ds(qq * 16, 16)
                s = er_t[sl]
                for t in range(1, 16):
                    s = s + er_t[pl.ds(t * NPT + qq * 16, 16)]
                den_t[sl] = 1.0 / jnp.maximum(s, 1e-38)
                return 0
            lax.fori_loop(0, NPT // 16, red, 0)
            pltpu.sync_copy(den_t.at[pl.ds(0, NPT)],
                            rden_sc.at[pl.ds(pl.multiple_of(cid * NP + stripe, K), NPT)])
            plsc.subcore_barrier()
            pltpu.sync_copy(rden_sc.at[pl.ds(pl.multiple_of(cid * NP, K), NP)], den_t)
            ph2_sweep(g, h)

        def zero_acc():
            pltpu.sync_copy(zeros128, acc_sh.at[pl.ds(sid * NPT, NPT)])
            plsc.subcore_barrier()

        def flush_acc(out_ref, head):
            plsc.subcore_barrier()
            pltpu.sync_copy(
                acc_sh.at[pl.ds(sid * NPT, NPT)],
                out_ref.at[pl.ds(pl.multiple_of(head * NP + sid * NPT, K), NPT)])
            plsc.subcore_barrier()

        def head_pass(hh, _):
            head = cid * 4 + hh
            zero_acc()

            def g_sweep(g, _):
                gat_head_pass(g, head)
                return 0
            lax.fori_loop(0, 3, g_sweep, 0)
            flush_acc(sum_s1, head)
            zero_acc()
            gat_head_pass(3, head)
            flush_acc(rst_f1, head)
            return 0

        lax.fori_loop(0, 4, head_pass, 0)

        def seg_scatter(f, base, n):
            if n == K:
                fo = pl.multiple_of(f * NS + base, 8)
                pltpu.sync_copy(feats_flat.at[pl.ds(fo, K // 2)], rowsA)
                pltpu.sync_copy(feats_flat.at[pl.ds(pl.multiple_of(fo + K // 2, 8), K // 2)],
                                rowsB)
                pltpu.sync_copy(
                    segs_all.at[pl.ds(pl.multiple_of(f * NP + base, K), K)], srcb)
                for j in range(4):
                    sl = pl.ds(j * 16, 16)
                    dstA[sl] = srcb[sl]
                    dstB[sl] = srcb[pl.ds((j + 4) * 16, 16)]
                pltpu.sync_copy(rowsA, acc_sh.at[dstA], add=True)
                pltpu.sync_copy(rowsB, acc_sh.at[dstB], add=True)
            else:
                pltpu.sync_copy(feats_flat.at[pl.ds(pl.multiple_of(f * NS + base, 8), n)],
                                rowsA.at[pl.ds(0, n)])
                pltpu.sync_copy(
                    segs_all.at[pl.ds(pl.multiple_of(f * NP + NBLKN * K, K), K)], srcb)
                idx16[...] = srcb[pl.ds(0, n)]
                pltpu.sync_copy(rowsA.at[pl.ds(0, n)], acc_sh.at[idx16], add=True)

        def seg_bcast(f, base, n):
            if n == K:
                pltpu.sync_copy(
                    segs_all.at[pl.ds(pl.multiple_of(f * NP + base, K), K)], srcb)
                for j in range(4):
                    sl = pl.ds(j * 16, 16)
                    dstA[sl] = srcb[sl] + f * B
                    dstB[sl] = srcb[pl.ds((j + 4) * 16, 16)] + f * B
                cpA = pltpu.async_copy(mean_sc.at[dstA], rowsA, sem)
                cpB = pltpu.async_copy(mean_sc.at[dstB], rowsB, sem2)
                cpA.wait()
                cpB.wait()
                fo = pl.multiple_of(f * NS + base, 8)
                pltpu.sync_copy(rowsA, meanb.at[pl.ds(fo, K // 2)])
                pltpu.sync_copy(rowsB,
                                meanb.at[pl.ds(pl.multiple_of(fo + K // 2, 8), K // 2)])
            else:
                pltpu.sync_copy(
                    segs_all.at[pl.ds(pl.multiple_of(f * NP + NBLKN * K, K), K)], srcb)
                idx16[...] = srcb[pl.ds(0, n)] + f * B
                pltpu.async_copy(mean_sc.at[idx16], rowsA.at[pl.ds(0, n)], sem).wait()
                pltpu.sync_copy(
                    rowsA.at[pl.ds(0, n)],
                    meanb.at[pl.ds(pl.multiple_of(f * NS + base, 8), n)])

        def node_sweep(fn):
            nb = (NBLKN - sid + 15) // 16

            def it(i, _):
                fn(pl.multiple_of((sid + i * 16) * K, K), K)
                return 0
            lax.fori_loop(0, nb, it, 0)

            @pl.when(sid == 0)
            def _():
                fn(NBLKN * K, NTAIL)

        def seg_mean(f, _):
            fcore = jnp.where(f == 0, 0, 1)

            @pl.when(cid == fcore)
            def _():
                pltpu.sync_copy(zeros128.at[pl.ds(0, 32)],
                                acc_sh.at[pl.ds(sid * 32, 32)])
                plsc.subcore_barrier()
                node_sweep(functools.partial(seg_scatter, f))
                plsc.subcore_barrier()
                pltpu.sync_copy(acc_sh.at[pl.ds(sid * 32, 32)],
                                rowsA.at[pl.ds(0, 32)])
                pltpu.sync_copy(
                    rcnt_all.at[pl.ds(pl.multiple_of(f * B + sid * 32, 8), 32)],
                    rowsA.at[pl.ds(32, 32)])

                def mrow(r, _):
                    for j in range(8):
                        sl = pl.ds(j * 16, 16)
                        rowsA[r, sl] = rowsA[r, sl] * rowsA[32 + r, sl]
                    return 0
                lax.fori_loop(0, 32, mrow, 0)
                pltpu.sync_copy(
                    rowsA.at[pl.ds(0, 32)],
                    mean_sc.at[pl.ds(pl.multiple_of(f * B + sid * 32, 8), 32)])
                plsc.subcore_barrier()
                node_sweep(functools.partial(seg_bcast, f))
            return 0

        lax.fori_loop(0, 3, seg_mean, 0)

    return body


def kernel(feat_s1, feat_f1, feat_c1, edge_intra, edge_inter, seg_s1, seg_f1, seg_c1, params):
    f32 = jnp.float32
    ps = [params["c1_intra"], params["c2_intra"], params["c2_inter"], params["c1_inter"]]
    xs_list = [feat_s1, feat_s1, feat_f1, feat_s1]
    xd_list = [feat_s1, feat_s1, feat_s1, feat_f1]

    hs_list, el_list, er_list = [], [], []
    for p, xs, xd in zip(ps, xs_list, xd_list):
        W3 = p["W"].reshape(D, H, D)
        Wl = jnp.einsum("khd,hd->kh", W3, p["al"])
        Wr = jnp.einsum("khd,hd->kh", W3, p["ar"])
        hs = (xs @ p["W"]).reshape(NS * H, D)
        hs_list.append(hs)
        el_list.append(jnp.pad(Wl.T @ xs.T, ((0, 0), (0, NP - NS))).reshape(-1))
        er_list.append(jnp.pad(Wr.T @ xd.T, ((0, 0), (0, NP - NS))).reshape(-1))

    hs_flat = jnp.concatenate(hs_list, axis=0)
    att_el = jnp.concatenate(el_list)
    att_er = jnp.concatenate(er_list)

    ei0, ei1 = edge_intra[0], edge_intra[1]
    eI0, eI1 = edge_inter[0], edge_inter[1]
    src_all = jnp.concatenate([ei0, ei1, eI1, eI0])
    dst_all = jnp.concatenate([ei1, ei0, eI0, eI1])
    edg_all = jnp.stack([src_all.reshape(NG * NBLKE, K),
                         dst_all.reshape(NG * NBLKE, K)], axis=1).reshape(-1)

    feats_flat = jnp.concatenate([feat_s1, feat_f1, feat_c1], axis=0)
    segs = [seg_s1, seg_f1, seg_c1]
    segs_all = jnp.concatenate([jnp.pad(sg, (0, NP - NS)) for sg in segs])
    rcnt_all = jnp.concatenate([
        jnp.tile((1.0 / jnp.maximum(
            jnp.zeros((B,), f32).at[sg].add(1.0), 1.0))[:, None], (1, D))
        for sg in segs])
    zeros128 = jnp.zeros((NPT, D), f32)

    sum_s1, rst_f1, meanb, _, _, _, _ = _sc_edge_kernel()(
        edg_all, hs_flat, att_el, att_er, feats_flat,
        segs_all, rcnt_all, zeros128)

    rW_s1 = ps[0]["rW"] + ps[1]["rW"] + ps[2]["rW"]
    b_s1 = ps[0]["b"] + ps[1]["b"] + ps[2]["b"]
    resid_s1 = (feat_s1 @ rW_s1 + b_s1).reshape(NS, H, D)
    resid_f1 = (feat_f1 @ ps[3]["rW"] + ps[3]["b"]).reshape(NS, H, D)

    sum_s1 = sum_s1.reshape(H, NP, D)[:, :NS]
    rst_f1 = rst_f1.reshape(H, NP, D)[:, :NS]
    h_s1 = jnp.max(sum_s1 + resid_s1.transpose(1, 0, 2), axis=0) + meanb[:NS]
    h_f1 = jnp.max(rst_f1 + resid_f1.transpose(1, 0, 2), axis=0) + meanb[NS:2 * NS]
    h_c1 = meanb[2 * NS:]
    return h_f1, h_c1, h_s1


# speedup vs baseline: 17.9808x; 1.0277x over previous
import functools

import jax
import jax.numpy as jnp
from jax import lax
from jax.experimental import pallas as pl
from jax.experimental.pallas import tpu as pltpu
from jax.experimental.pallas import tpu_sc as plsc

H = 8
D = 128
B = 512
NS = 10000
E = 160000
NG = 4
K = 128
NBLKE = E // K
NP = 10240
NPT = NP // 16
NBLKN = NS // K
NTAIL = NS - NBLKN * K
HS_ROWS = NS * H


def _sc_edge_kernel():
    mesh = plsc.VectorSubcoreMesh(core_axis_name="c", subcore_axis_name="s")
    f32 = jnp.float32
    i32 = jnp.int32

    out_type = [
        jax.ShapeDtypeStruct((H * NP, D), f32),
        jax.ShapeDtypeStruct((H * NP, D), f32),
        jax.ShapeDtypeStruct((3 * NS, D), f32),
        jax.ShapeDtypeStruct((2 * 16 * NP,), f32),
        jax.ShapeDtypeStruct((2 * NP,), f32),
        jax.ShapeDtypeStruct((2 * E,), f32),
        jax.ShapeDtypeStruct((3 * B, D), f32),
    ]
    scratch_types = [
        pltpu.VMEM((K,), i32),
        pltpu.VMEM((K,), i32),
        pltpu.VMEM((K,), i32),
        pltpu.VMEM((K,), f32),
        pltpu.VMEM((NP,), f32),
        pltpu.VMEM((NP,), f32),
        pltpu.VMEM((NP,), f32),
        pltpu.VMEM((K // 2, D), f32),
        pltpu.VMEM((K // 2, D), f32),
        pltpu.VMEM((K // 2,), i32),
        pltpu.VMEM((K // 2,), i32),
        pltpu.VMEM((K // 2,), i32),
        pltpu.VMEM((K // 2,), i32),
        pltpu.VMEM((16,), i32),
        pltpu.VMEM((2 * K,), i32),
        pltpu.VMEM((2 * K,), i32),
        pltpu.VMEM((K,), f32),
        pltpu.VMEM_SHARED((NP, D), f32),
        pltpu.SemaphoreType.DMA,
        pltpu.SemaphoreType.DMA,
        pltpu.SemaphoreType.DMA,
        pltpu.SemaphoreType.DMA,
        pltpu.SemaphoreType.DMA,
    ]

    @functools.partial(
        pl.kernel, out_type=out_type, mesh=mesh, scratch_types=scratch_types,
        compiler_params=pltpu.CompilerParams(needs_layout_passes=False))
    def body(edg_all, hs_flat, att_el, att_er, feats_flat,
             segs_all, rcnt_all, zeros128,
             sum_s1, rst_f1, meanb, den_part, rden_sc, ex_sc, mean_sc,
             srcb, dstb, idxb, exh, el_t, er_t, den_t, rowsA, rowsB,
             idxA, idxB, dstA, dstB, idx16, edg0, edg1, exh1,
             acc_sh, sem, sem2, semP0, semP1, semS):
        cid = lax.axis_index("c")
        sid = lax.axis_index("s")

        def edge_sweep(chunk_fn):
            nb = (NBLKE - sid + 15) // 16

            def it(i, _):
                chunk_fn(pl.multiple_of((sid + i * 16) * K, K))
                return 0
            lax.fori_loop(0, nb, it, 0)

        def ph1_prefetch(g, blk, edgb, semP):
            eo = pl.multiple_of(2 * g * E + blk * (2 * K), 2 * K)
            pltpu.async_copy(edg_all.at[pl.ds(eo, 2 * K)], edgb, semP)

        def ph1_process(g, base, edgb):
            def q(j, _):
                sl = pl.ds(j * 16, 16)
                dv = edgb[pl.ds(K + j * 16, 16)]
                z = (plsc.load_gather(el_t, [edgb[sl]])
                     + plsc.load_gather(er_t, [dv]))
                z = jnp.where(z >= 0.0, z, 0.2 * z)
                ex = jnp.exp(jnp.minimum(z, 60.0))
                exh[sl] = ex
                plsc.addupdate_scatter(den_t, [dv], ex)
                return 0
            lax.fori_loop(0, K // 16, q, 0)
            pltpu.sync_copy(exh, ex_sc.at[pl.ds(pl.multiple_of(cid * E + base, K), K)])

        def ph1_sweep(g, h):
            nb = (NBLKE - sid + 15) // 16
            ph1_prefetch(g, sid, edg0, semP0)

            def pair(m, _):
                i0 = 2 * m

                @pl.when(i0 + 1 < nb)
                def _():
                    ph1_prefetch(g, sid + (i0 + 1) * 16, edg1, semP1)
                pltpu.make_async_copy(edg_all.at[pl.ds(0, 2 * K)], edg0, semP0).wait()
                ph1_process(g, (sid + i0 * 16) * K, edg0)

                @pl.when(i0 + 1 < nb)
                def _():
                    @pl.when(i0 + 2 < nb)
                    def _():
                        ph1_prefetch(g, sid + (i0 + 2) * 16, edg0, semP0)
                    pltpu.make_async_copy(edg_all.at[pl.ds(0, 2 * K)], edg1, semP1).wait()
                    ph1_process(g, (sid + (i0 + 1) * 16) * K, edg1)
                return 0
            lax.fori_loop(0, (nb + 1) // 2, pair, 0)

        def ph2_prefetch(g, blk, edgb, exhb, semP):
            eo = pl.multiple_of(2 * g * E + blk * (2 * K), 2 * K)
            xo = pl.multiple_of(cid * E + blk * K, K)
            c1 = pltpu.async_copy(edg_all.at[pl.ds(eo, 2 * K)], edgb, semP)
            c2 = pltpu.async_copy(ex_sc.at[pl.ds(xo, K)], exhb, semP)
            return c1, c2

        def ph2_process(g, head, edgb, exhb):
            for j in range(4):
                sl = pl.ds(j * 16, 16)
                slb = pl.ds((j + 4) * 16, 16)
                idxA[sl] = edgb[sl] * H + (g * HS_ROWS + head)
                idxB[sl] = edgb[slb] * H + (g * HS_ROWS + head)
                dstA[sl] = edgb[pl.ds(K + j * 16, 16)]
                dstB[sl] = edgb[pl.ds(K + (j + 4) * 16, 16)]
            cpA = pltpu.async_copy(hs_flat.at[idxA], rowsA, sem)
            cpB = pltpu.async_copy(hs_flat.at[idxB], rowsB, sem2)

            def q(j, _):
                sl = pl.ds(j * 16, 16)
                exhb[sl] = exhb[sl] * plsc.load_gather(
                    den_t, [edgb[pl.ds(K + j * 16, 16)]])
                return 0
            lax.fori_loop(0, K // 16, q, 0)

            def scale(buf, eoff):
                def rowq(qq, _):
                    avq = exhb[pl.ds(pl.multiple_of(eoff + qq * 16, 16), 16)]
                    for l in range(16):
                        av = jnp.take(avq, jnp.full((16,), l, i32))
                        k = qq * 16 + l
                        for j in range(8):
                            sl = pl.ds(j * 16, 16)
                            buf[k, sl] = buf[k, sl] * av
                    return 0
                lax.fori_loop(0, K // 32, rowq, 0)

            cpA.wait()
            scale(rowsA, 0)
            scA = pltpu.async_copy(rowsA, acc_sh.at[dstA], add=True, sem=semS)
            cpB.wait()
            scale(rowsB, K // 2)
            scA.wait()
            pltpu.sync_copy(rowsB, acc_sh.at[dstB], add=True)

        def ph2_sweep(g, head):
            nb = (NBLKE - sid + 15) // 16
            pf0 = ph2_prefetch(g, sid, edg0, exh, semP0)

            def pair(m, _):
                i0 = 2 * m

                @pl.when(i0 + 1 < nb)
                def _():
                    ph2_prefetch(g, sid + (i0 + 1) * 16, edg1, exh1, semP1)
                pltpu.make_async_copy(edg_all.at[pl.ds(0, 2 * K)], edg0, semP0).wait()
                pltpu.make_async_copy(ex_sc.at[pl.ds(0, K)], exh, semP0).wait()
                ph2_process(g, head, edg0, exh)

                @pl.when(i0 + 1 < nb)
                def _():
                    @pl.when(i0 + 2 < nb)
                    def _():
                        ph2_prefetch(g, sid + (i0 + 2) * 16, edg0, exh, semP0)
                    pltpu.make_async_copy(edg_all.at[pl.ds(0, 2 * K)], edg1, semP1).wait()
                    pltpu.make_async_copy(ex_sc.at[pl.ds(0, K)], exh1, semP1).wait()
                    ph2_process(g, head, edg1, exh1)
                return 0
            lax.fori_loop(0, (nb + 1) // 2, pair, 0)

        def gat_head_pass(g, h):
            off = pl.multiple_of((g * H + h) * NP, K)
            pltpu.sync_copy(att_el.at[pl.ds(off, NP)], el_t)
            pltpu.sync_copy(att_er.at[pl.ds(off, NP)], er_t)

            def z16(i, _):
                den_t[pl.ds(i * 16, 16)] = jnp.zeros((16,), f32)
                return 0
            lax.fori_loop(0, NP // 16, z16, 0)
            ph1_sweep(g, h)
            pltpu.sync_copy(
                den_t, den_part.at[pl.ds(pl.multiple_of((cid * 16 + sid) * NP, K), NP)])
            plsc.subcore_barrier()
            stripe = pl.multiple_of(sid * NPT, K)

            def ld(t, _):
                pltpu.async_copy(
                    den_part.at[pl.ds(pl.multiple_of((cid * 16 + t) * NP + stripe, K), NPT)],
                    er_t.at[pl.ds(t * NPT, NPT)], semP0)
                return 0
            lax.fori_loop(0, 16, ld, 0)

            def lddrain(t, _):
                pltpu.make_async_copy(
                    den_part.at[pl.ds(0, NPT)],
                    er_t.at[pl.ds(t * NPT, NPT)], semP0).wait()
                return 0
            lax.fori_loop(0, 16, lddrain, 0)

            def red(qq, _):
                sl = pl.ds(qq * 16, 16)
                s = er_t[sl]
                for t in range(1, 16):
                    s = s + er_t[pl.ds(t * NPT + qq * 16, 16)]
                den_t[sl] = 1.0 / jnp.maximum(s, 1e-38)
                return 0
            lax.fori_loop(0, NPT // 16, red, 0)
            pltpu.sync_copy(den_t.at[pl.ds(0, NPT)],
                            rden_sc.at[pl.ds(pl.multiple_of(cid * NP + stripe, K), NPT)])
            plsc.subcore_barrier()
            pltpu.sync_copy(rden_sc.at[pl.ds(pl.multiple_of(cid * NP, K), NP)], den_t)
            ph2_sweep(g, h)

        def zero_acc():
            pltpu.sync_copy(zeros128, acc_sh.at[pl.ds(sid * NPT, NPT)])
            plsc.subcore_barrier()

        def flush_acc(out_ref, head):
            plsc.subcore_barrier()
            pltpu.sync_copy(
                acc_sh.at[pl.ds(sid * NPT, NPT)],
                out_ref.at[pl.ds(pl.multiple_of(head * NP + sid * NPT, K), NPT)])
            plsc.subcore_barrier()

        def head_pass(hh, _):
            head = cid * 4 + hh
            zero_acc()

            def g_sweep(g, _):
                gat_head_pass(g, head)
                return 0
            lax.fori_loop(0, 3, g_sweep, 0)
            flush_acc(sum_s1, head)
            zero_acc()
            gat_head_pass(3, head)
            flush_acc(rst_f1, head)
            return 0

        lax.fori_loop(0, 4, head_pass, 0)

        def seg_scatter(f, base, n):
            if n == K:
                fo = pl.multiple_of(f * NS + base, 8)
                pltpu.sync_copy(feats_flat.at[pl.ds(fo, K // 2)], rowsA)
                pltpu.sync_copy(feats_flat.at[pl.ds(pl.multiple_of(fo + K // 2, 8), K // 2)],
                                rowsB)
                pltpu.sync_copy(
                    segs_all.at[pl.ds(pl.multiple_of(f * NP + base, K), K)], srcb)
                for j in range(4):
                    sl = pl.ds(j * 16, 16)
                    dstA[sl] = srcb[sl]
                    dstB[sl] = srcb[pl.ds((j + 4) * 16, 16)]
                pltpu.sync_copy(rowsA, acc_sh.at[dstA], add=True)
                pltpu.sync_copy(rowsB, acc_sh.at[dstB], add=True)
            else:
                pltpu.sync_copy(feats_flat.at[pl.ds(pl.multiple_of(f * NS + base, 8), n)],
                                rowsA.at[pl.ds(0, n)])
                pltpu.sync_copy(
                    segs_all.at[pl.ds(pl.multiple_of(f * NP + NBLKN * K, K), K)], srcb)
                idx16[...] = srcb[pl.ds(0, n)]
                pltpu.sync_copy(rowsA.at[pl.ds(0, n)], acc_sh.at[idx16], add=True)

        def seg_bcast(f, base, n):
            if n == K:
                pltpu.sync_copy(
                    segs_all.at[pl.ds(pl.multiple_of(f * NP + base, K), K)], srcb)
                for j in range(4):
                    sl = pl.ds(j * 16, 16)
                    dstA[sl] = srcb[sl] + f * B
                    dstB[sl] = srcb[pl.ds((j + 4) * 16, 16)] + f * B
                cpA = pltpu.async_copy(mean_sc.at[dstA], rowsA, sem)
                cpB = pltpu.async_copy(mean_sc.at[dstB], rowsB, sem2)
                cpA.wait()
                cpB.wait()
                fo = pl.multiple_of(f * NS + base, 8)
                pltpu.sync_copy(rowsA, meanb.at[pl.ds(fo, K // 2)])
                pltpu.sync_copy(rowsB,
                                meanb.at[pl.ds(pl.multiple_of(fo + K // 2, 8), K // 2)])
            else:
                pltpu.sync_copy(
                    segs_all.at[pl.ds(pl.multiple_of(f * NP + NBLKN * K, K), K)], srcb)
                idx16[...] = srcb[pl.ds(0, n)] + f * B
                pltpu.async_copy(mean_sc.at[idx16], rowsA.at[pl.ds(0, n)], sem).wait()
                pltpu.sync_copy(
                    rowsA.at[pl.ds(0, n)],
                    meanb.at[pl.ds(pl.multiple_of(f * NS + base, 8), n)])

        def node_sweep(fn):
            nb = (NBLKN - sid + 15) // 16

            def it(i, _):
                fn(pl.multiple_of((sid + i * 16) * K, K), K)
                return 0
            lax.fori_loop(0, nb, it, 0)

            @pl.when(sid == 0)
            def _():
                fn(NBLKN * K, NTAIL)

        def seg_mean(f, _):
            fcore = jnp.where(f == 0, 0, 1)

            @pl.when(cid == fcore)
            def _():
                pltpu.sync_copy(zeros128.at[pl.ds(0, 32)],
                                acc_sh.at[pl.ds(sid * 32, 32)])
                plsc.subcore_barrier()
                node_sweep(functools.partial(seg_scatter, f))
                plsc.subcore_barrier()
                pltpu.sync_copy(acc_sh.at[pl.ds(sid * 32, 32)],
                                rowsA.at[pl.ds(0, 32)])
                pltpu.sync_copy(
                    rcnt_all.at[pl.ds(pl.multiple_of(f * B + sid * 32, 8), 32)],
                    rowsA.at[pl.ds(32, 32)])

                def mrow(r, _):
                    for j in range(8):
                        sl = pl.ds(j * 16, 16)
                        rowsA[r, sl] = rowsA[r, sl] * rowsA[32 + r, sl]
                    return 0
                lax.fori_loop(0, 32, mrow, 0)
                pltpu.sync_copy(
                    rowsA.at[pl.ds(0, 32)],
                    mean_sc.at[pl.ds(pl.multiple_of(f * B + sid * 32, 8), 32)])
                plsc.subcore_barrier()
                node_sweep(functools.partial(seg_bcast, f))
            return 0

        lax.fori_loop(0, 3, seg_mean, 0)

    return body


def kernel(feat_s1, feat_f1, feat_c1, edge_intra, edge_inter, seg_s1, seg_f1, seg_c1, params):
    f32 = jnp.float32
    ps = [params["c1_intra"], params["c2_intra"], params["c2_inter"], params["c1_inter"]]
    xs_list = [feat_s1, feat_s1, feat_f1, feat_s1]
    xd_list = [feat_s1, feat_s1, feat_s1, feat_f1]

    hs_list, el_list, er_list = [], [], []
    for p, xs, xd in zip(ps, xs_list, xd_list):
        W3 = p["W"].reshape(D, H, D)
        Wl = jnp.einsum("khd,hd->kh", W3, p["al"])
        Wr = jnp.einsum("khd,hd->kh", W3, p["ar"])
        hs = (xs @ p["W"]).reshape(NS * H, D)
        hs_list.append(hs)
        el_list.append(jnp.pad(Wl.T @ xs.T, ((0, 0), (0, NP - NS))).reshape(-1))
        er_list.append(jnp.pad(Wr.T @ xd.T, ((0, 0), (0, NP - NS))).reshape(-1))

    hs_flat = jnp.concatenate(hs_list, axis=0)
    att_el = jnp.concatenate(el_list)
    att_er = jnp.concatenate(er_list)

    ei0, ei1 = edge_intra[0], edge_intra[1]
    eI0, eI1 = edge_inter[0], edge_inter[1]
    src_all = jnp.concatenate([ei0, ei1, eI1, eI0])
    dst_all = jnp.concatenate([ei1, ei0, eI0, eI1])
    edg_all = jnp.stack([src_all.reshape(NG * NBLKE, K),
                         dst_all.reshape(NG * NBLKE, K)], axis=1).reshape(-1)

    feats_flat = jnp.concatenate([feat_s1, feat_f1, feat_c1], axis=0)
    segs = [seg_s1, seg_f1, seg_c1]
    segs_all = jnp.concatenate([jnp.pad(sg, (0, NP - NS)) for sg in segs])
    rcnt_all = jnp.concatenate([
        jnp.tile((1.0 / jnp.maximum(
            jnp.zeros((B,), f32).at[sg].add(1.0), 1.0))[:, None], (1, D))
        for sg in segs])
    zeros128 = jnp.zeros((NPT, D), f32)

    sum_s1, rst_f1, meanb, _, _, _, _ = _sc_edge_kernel()(
        edg_all, hs_flat, att_el, att_er, feats_flat,
        segs_all, rcnt_all, zeros128)

    rW_s1 = ps[0]["rW"] + ps[1]["rW"] + ps[2]["rW"]
    b_s1 = ps[0]["b"] + ps[1]["b"] + ps[2]["b"]
    resid_s1 = (feat_s1 @ rW_s1 + b_s1).reshape(NS, H, D)
    resid_f1 = (feat_f1 @ ps[3]["rW"] + ps[3]["b"]).reshape(NS, H, D)

    sum_s1 = sum_s1.reshape(H, NP, D)[:, :NS]
    rst_f1 = rst_f1.reshape(H, NP, D)[:, :NS]
    h_s1 = jnp.max(sum_s1 + resid_s1.transpose(1, 0, 2), axis=0) + meanb[:NS]
    h_f1 = jnp.max(rst_f1 + resid_f1.transpose(1, 0, 2), axis=0) + meanb[NS:2 * NS]
    h_c1 = meanb[2 * NS:]
    return h_f1, h_c1, h_s1
